# Initial kernel scaffold; baseline (speedup 1.0000x reference)
import jax
import jax.numpy as jnp
from jax.experimental import pallas as pl


def kernel(x, edge_index, edge_attrs, batch, params):
    raise NotImplementedError("write your pallas kernel here")


# speedup vs baseline: 1.0925x; 1.0925x over previous
import functools

import jax
import jax.numpy as jnp
from jax import lax
from jax.experimental import pallas as pl
from jax.experimental.pallas import tpu as pltpu
from jax.experimental.pallas import tpu_sc as plsc

N = 10000
E = 160000
B = 64
DF = 128
DE = 16
H = 16
EPS = 1e-5

NC, NS = 2, 16
NW = NC * NS
CH = 128
EPW = 5120
E_PAD = NW * EPW
NCHUNK = EPW // CH
NROW_T = 626
N_PAD = NS * NROW_T

_f32 = jnp.float32
_HI = lax.Precision.HIGHEST
_i32 = jnp.int32


def _sc_gather_body(h_hbm, src_hbm, out_hbm, idx_v, rows_v, sem):
    w = lax.axis_index("c") * NS + lax.axis_index("s")
    pltpu.sync_copy(src_hbm.at[pl.ds(w * NCHUNK, NCHUNK)], idx_v)

    def chunk(j, carry):
        pltpu.async_copy(
            h_hbm.at[idx_v.at[j]], rows_v.at[pl.ds(j * CH, CH)], sem
        ).wait()
        return carry

    lax.fori_loop(0, NCHUNK, chunk, 0, unroll=8)
    pltpu.sync_copy(rows_v, out_hbm.at[pl.ds(w * EPW, EPW)])


@functools.partial(jax.jit, static_argnames=())
def _sc_gather(h, src2):
    f = pl.kernel(
        _sc_gather_body,
        out_type=jax.ShapeDtypeStruct((E_PAD, H), _f32),
        mesh=plsc.VectorSubcoreMesh(core_axis_name="c", subcore_axis_name="s"),
        scratch_types=[
            pltpu.VMEM((NCHUNK, CH), _i32),
            pltpu.VMEM((EPW, H), _f32),
            pltpu.SemaphoreType.DMA,
        ],
        compiler_params=pltpu.CompilerParams(use_tc_tiling_on_sc=False),
    )
    return f(h, src2)


def _sc_scatter_body(msg_hbm, dst_hbm, zero_hbm, out_hbm, idx_v, msg_v, acc_sh):
    c = lax.axis_index("c")
    s = lax.axis_index("s")
    w = c * NS + s
    pltpu.sync_copy(zero_hbm, acc_sh.at[pl.ds(s * NROW_T, NROW_T)])
    pltpu.sync_copy(msg_hbm.at[pl.ds(w * EPW, EPW)], msg_v)
    pltpu.sync_copy(dst_hbm.at[pl.ds(w * NCHUNK, NCHUNK)], idx_v)
    plsc.subcore_barrier()

    def chunk(j, carry):
        pltpu.sync_copy(
            msg_v.at[pl.ds(j * CH, CH)], acc_sh.at[idx_v.at[j]], add=True
        )
        return carry

    lax.fori_loop(0, NCHUNK, chunk, 0, unroll=8)
    plsc.subcore_barrier()
    pltpu.sync_copy(
        acc_sh.at[pl.ds(s * NROW_T, NROW_T)],
        out_hbm.at[c, pl.ds(s * NROW_T, NROW_T)],
    )


def _sc_scatter(msg, dst2, zero_blk):
    f = pl.kernel(
        _sc_scatter_body,
        out_type=jax.ShapeDtypeStruct((NC, N_PAD, H), _f32),
        mesh=plsc.VectorSubcoreMesh(core_axis_name="c", subcore_axis_name="s"),
        scratch_types=[
            pltpu.VMEM((NCHUNK, CH), _i32),
            pltpu.VMEM((EPW, H), _f32),
            pltpu.VMEM_SHARED((N_PAD, H), _f32),
        ],
        compiler_params=pltpu.CompilerParams(use_tc_tiling_on_sc=False),
    )
    return f(msg, dst2, zero_blk)


def _bn_rows(y, g, b):
    m = jnp.mean(y, axis=0, keepdims=True)
    v = jnp.mean((y - m) * (y - m), axis=0, keepdims=True)
    return (y - m) * lax.rsqrt(v + EPS) * g + b


def _embed_h_body(x_ref, w_ref, b_ref, g_ref, be_ref, o_ref):
    y = jnp.dot(x_ref[...], w_ref[...], preferred_element_type=_f32, precision=_HI) + b_ref[...]
    o_ref[...] = _bn_rows(y, g_ref[...], be_ref[...])


def _embed_y_body(ea_ref, w_ref, b_ref, y_ref, st_ref, acc_ref):
    i = pl.program_id(0)
    y = jnp.dot(ea_ref[...], w_ref[...], preferred_element_type=_f32, precision=_HI) + b_ref[...]
    y_ref[...] = y
    s1 = jnp.sum(y, axis=0, keepdims=True)
    s2 = jnp.sum(y * y, axis=0, keepdims=True)
    blk = jnp.concatenate([s1, s2], axis=0)

    @pl.when(i == 0)
    def _():
        acc_ref[...] = blk

    @pl.when(i > 0)
    def _():
        acc_ref[...] = acc_ref[...] + blk

    @pl.when(i == pl.num_programs(0) - 1)
    def _():
        st_ref[...] = acc_ref[...]


def _msg_body(y_ref, hs_ref, wa_ref, c_ref, o_ref):
    w = jnp.dot(y_ref[...], wa_ref[...], preferred_element_type=_f32,
                precision=_HI) + c_ref[...]
    hb = hs_ref[...]
    msg = hb[:, 0:1] * w[:, 0:H]
    for i in range(1, H):
        msg = msg + hb[:, i : i + 1] * w[:, i * H : (i + 1) * H]
    o_ref[...] = msg


def _update_body(h_ref, agg_ref, wr_ref, bc_ref, g_ref, be_ref, o_ref):
    a = agg_ref[0, : N, :] + agg_ref[1, : N, :]
    y = jnp.dot(h_ref[...], wr_ref[...], preferred_element_type=_f32, precision=_HI)
    y = jnp.maximum(y + a + bc_ref[...], 0.0)
    o_ref[...] = _bn_rows(y, g_ref[...], be_ref[...])


def _pool_mlp_body(h_ref, b_ref, ones_ref, wp0, bp0, gp0, bep0,
                   wp1, bp1, gp1, bep1, wp2, bp2, gp2, bep2, wp3, bp3,
                   o_ref, mx_ref):
    h = h_ref[...]
    bb = b_ref[...]
    seg = lax.broadcasted_iota(_i32, (N, B), 1)
    oh = (bb == seg).astype(_f32)
    dn = (((0,), (0,)), ((), ()))
    sums = lax.dot_general(oh, h, dn, preferred_element_type=_f32, precision=_HI)
    cnt = lax.dot_general(oh, ones_ref[...], dn, preferred_element_type=_f32, precision=_HI)
    mean = sums / jnp.maximum(cnt, 1.0)
    neg = jnp.float32(-jnp.inf)

    def seg_max(b, carry):
        masked = jnp.where(bb == b, h, neg)
        mx_ref[pl.ds(b, 1), :] = jnp.max(masked, axis=0, keepdims=True)
        return carry

    lax.fori_loop(0, B, seg_max, 0)
    mx = mx_ref[...]
    mx = jnp.where(jnp.isfinite(mx), mx, 0.0)
    z = jnp.concatenate([mean, mx], axis=1)
    for wp, bp, gp, bep in ((wp0, bp0, gp0, bep0), (wp1, bp1, gp1, bep1),
                            (wp2, bp2, gp2, bep2)):
        z = jnp.maximum(jnp.dot(z, wp[...], preferred_element_type=_f32, precision=_HI)
                        + bp[...], 0.0)
        z = _bn_rows(z, gp[...], bep[...])
    o_ref[...] = jnp.dot(z, wp3[...], preferred_element_type=_f32, precision=_HI) + bp3[...]


_EBLK = 2048
_YBLK = 8000


def kernel(x, edge_index, edge_attrs, batch, params):
    p = params
    src = edge_index[0].astype(_i32)
    dst = edge_index[1].astype(_i32)
    src2 = jnp.concatenate([src, jnp.zeros((E_PAD - E,), _i32)]).reshape(-1, CH)
    dst2 = jnp.concatenate([dst, jnp.full((E_PAD - E,), N, _i32)]).reshape(-1, CH)
    batch_c = batch.astype(_i32).reshape(N, 1)

    row16 = lambda a: a.reshape(1, -1).astype(_f32)

    h = pl.pallas_call(
        _embed_h_body,
        out_shape=jax.ShapeDtypeStruct((N, H), _f32),
    )(x, p["Wne"], row16(p["bne"]), row16(p["gne"]), row16(p["bene"]))

    n_yblk = E // _YBLK
    y, stats = pl.pallas_call(
        _embed_y_body,
        grid=(n_yblk,),
        in_specs=[
            pl.BlockSpec((_YBLK, DE), lambda i: (i, 0)),
            pl.BlockSpec((DE, H), lambda i: (0, 0)),
            pl.BlockSpec((1, H), lambda i: (0, 0)),
        ],
        out_specs=[
            pl.BlockSpec((_YBLK, H), lambda i: (i, 0)),
            pl.BlockSpec((2, H), lambda i: (0, 0)),
        ],
        out_shape=[
            jax.ShapeDtypeStruct((E, H), _f32),
            jax.ShapeDtypeStruct((2, H), _f32),
        ],
        scratch_shapes=[pltpu.VMEM((2, H), _f32)],
    )(edge_attrs, p["Wee"], row16(p["bee"]))

    m = stats[0] / E
    v = stats[1] / E - m * m
    alpha = p["gee"] * lax.rsqrt(v + EPS)
    beta = p["beee"] - m * alpha

    y_pad = jnp.concatenate([y, jnp.zeros((E_PAD - E, H), _f32)], axis=0)
    zero_blk = jnp.zeros((NROW_T, H), _f32)

    n_eblk = E_PAD // _EBLK
    msg_call = pl.pallas_call(
        _msg_body,
        grid=(n_eblk,),
        in_specs=[
            pl.BlockSpec((_EBLK, H), lambda i: (i, 0)),
            pl.BlockSpec((_EBLK, H), lambda i: (i, 0)),
            pl.BlockSpec((H, H * H), lambda i: (0, 0)),
            pl.BlockSpec((1, H * H), lambda i: (0, 0)),
        ],
        out_specs=pl.BlockSpec((_EBLK, H), lambda i: (i, 0)),
        out_shape=jax.ShapeDtypeStruct((E_PAD, H), _f32),
    )

    update_call = pl.pallas_call(
        _update_body,
        out_shape=jax.ShapeDtypeStruct((N, H), _f32),
    )

    for i in range(5):
        wnn = p["Wnn%d" % i]
        wnn_a = alpha[:, None] * wnn
        c_row = (beta @ wnn + p["bnn%d" % i]).reshape(1, H * H)
        hs = _sc_gather(h, src2)
        msg = msg_call(y_pad, hs, wnn_a, c_row)
        aggp = _sc_scatter(msg, dst2, zero_blk)
        h = update_call(h, aggp, p["Wroot%d" % i], row16(p["bc%d" % i]),
                        row16(p["g%d" % i]), row16(p["be%d" % i]))

    ones_col = jnp.ones((N, 1), _f32)
    out = pl.pallas_call(
        _pool_mlp_body,
        out_shape=jax.ShapeDtypeStruct((B, 1), _f32),
        scratch_shapes=[pltpu.VMEM((B, H), _f32)],
    )(h, batch_c, ones_col,
      p["Wp0"], row16(p["bp0"]), row16(p["gp0"]), row16(p["bep0"]),
      p["Wp1"], row16(p["bp1"]), row16(p["gp1"]), row16(p["bep1"]),
      p["Wp2"], row16(p["bp2"]), row16(p["gp2"]), row16(p["bep2"]),
      p["Wp3"], row16(p["bp3"]))
    return out
